# baseline (speedup 1.0000x reference)
import functools

import jax
import jax.numpy as jnp
from jax.experimental import pallas as pl
from jax.experimental.pallas import tpu as pltpu

N_DRUG = 10000
N_DIS = 10000
H = 128


def _decoder_matmul_kernel(a_ref, b_ref, o_ref):
    o_ref[...] = jax.lax.dot_general(
        a_ref[...], b_ref[...], (((1,), (1,)), ((), ())),
        preferred_element_type=jnp.float32)


def _decoder_matmul(a, b, bm=512, bn=512):
    m, k = a.shape
    n = b.shape[0]
    grid = (pl.cdiv(m, bm), pl.cdiv(n, bn))
    return pl.pallas_call(
        _decoder_matmul_kernel,
        grid=grid,
        in_specs=[
            pl.BlockSpec((bm, k), lambda i, j: (i, 0)),
            pl.BlockSpec((bn, k), lambda i, j: (j, 0)),
        ],
        out_specs=pl.BlockSpec((bm, bn), lambda i, j: (i, j)),
        out_shape=jax.ShapeDtypeStruct((m, n), jnp.float32),
    )(a, b)


def _graph_conv(x_src, src, dst, n_src, n_dst, W, b):
    deg_src = jnp.maximum(jnp.zeros((n_src,), jnp.float32).at[src].add(1.0), 1.0)
    deg_dst = jnp.maximum(jnp.zeros((n_dst,), jnp.float32).at[dst].add(1.0), 1.0)
    h = x_src * jax.lax.rsqrt(deg_src)[:, None]
    agg = jnp.zeros((n_dst, x_src.shape[1]), x_src.dtype).at[dst].add(h[src])
    agg = agg * jax.lax.rsqrt(deg_dst)[:, None]
    return agg @ W.T + b


def _bn_prelu(v, gamma, beta, a):
    v = gamma * v + beta
    return jnp.where(v >= 0, v, a * v)


def _sem_att(z, W1, b1, w2):
    w = jnp.tanh(z @ W1.T + b1) @ w2
    beta = jax.nn.softmax(w.mean(0))
    return (beta[None, :, None] * z).sum(1)


def kernel(x_drug, x_disease, edge_dr2di, edge_di2dr,
           W_drug_lin, b_drug_lin, W_dis_lin, b_dis_lin,
           e1_W_dr2di, e1_b_dr2di, e1_W_di2dr, e1_b_di2dr, e1_gamma, e1_beta, e1_prelu,
           e2_W_dr2di, e2_b_dr2di, e2_W_di2dr, e2_b_di2dr, e2_gamma, e2_beta, e2_prelu,
           att_dr_W1, att_dr_b1, att_dr_w2, att_di_W1, att_di_b1, att_di_w2,
           W_R, W_D):
    h_dr = x_drug @ W_drug_lin.T + b_drug_lin
    h_di = x_disease @ W_dis_lin.T + b_dis_lin
    dr_emb = [h_dr]
    di_emb = [h_di]
    new_di = _graph_conv(h_dr, edge_dr2di[0], edge_dr2di[1], N_DRUG, N_DIS, e1_W_dr2di, e1_b_dr2di)
    new_dr = _graph_conv(h_di, edge_di2dr[0], edge_di2dr[1], N_DIS, N_DRUG, e1_W_di2dr, e1_b_di2dr)
    h_dr = _bn_prelu(new_dr, e1_gamma, e1_beta, e1_prelu)
    h_di = _bn_prelu(new_di, e1_gamma, e1_beta, e1_prelu)
    dr_emb.append(h_dr)
    di_emb.append(h_di)
    new_di = _graph_conv(h_dr, edge_dr2di[0], edge_dr2di[1], N_DRUG, N_DIS, e2_W_dr2di, e2_b_dr2di)
    new_dr = _graph_conv(h_di, edge_di2dr[0], edge_di2dr[1], N_DIS, N_DRUG, e2_W_di2dr, e2_b_di2dr)
    h_dr = _bn_prelu(new_dr, e2_gamma, e2_beta, e2_prelu)
    h_di = _bn_prelu(new_di, e2_gamma, e2_beta, e2_prelu)
    dr_emb.append(h_dr)
    di_emb.append(h_di)
    z_dr = jnp.stack(dr_emb, axis=1)
    z_di = jnp.stack(di_emb, axis=1)
    drug_f = _sem_att(z_dr, att_dr_W1, att_dr_b1, att_dr_w2)
    dis_f = _sem_att(z_di, att_di_W1, att_di_b1, att_di_w2)
    M = W_R.T @ W_D
    a = drug_f @ M
    return _decoder_matmul(a, dis_f)


# speedup vs baseline: 2.8787x; 2.8787x over previous
import functools

import jax
import jax.numpy as jnp
from jax import lax
from jax.experimental import pallas as pl
from jax.experimental.pallas import tpu as pltpu
from jax.experimental.pallas import tpu_sc as plsc

N_DRUG = 10000
N_DIS = 10000
N_ALL = N_DRUG + N_DIS
E = 320000
H = 128

NC = 2
NS = 16
L = 16

HIST = 20480
_DEG_PER_CORE = 2 * E
_DEG_PER_SUB = _DEG_PER_CORE // NS
_DEG_C = 80
_DEG_ITERS = _DEG_PER_SUB // _DEG_C
_HIST_PER_SUB = HIST // NS


def _sc_degree_body(idx_hbm, out_hbm, hist_sp, idx_v, ones_v, zero_v):
    c = lax.axis_index("c")
    s = lax.axis_index("s")
    zeros16 = jnp.zeros((L,), jnp.float32)
    ones16 = jnp.ones((L,), jnp.float32)
    def fill_zero(i, _):
        zero_v[pl.ds(i * L, L)] = zeros16
        return 0
    lax.fori_loop(0, _HIST_PER_SUB // L, fill_zero, 0)
    for k in range(_DEG_C // L):
        ones_v[pl.ds(k * L, L)] = ones16
    pltpu.sync_copy(zero_v, hist_sp.at[pl.ds(s * _HIST_PER_SUB, _HIST_PER_SUB)])
    plsc.subcore_barrier()

    base0 = c * _DEG_PER_CORE + s * _DEG_PER_SUB

    def body(i, _):
        pltpu.sync_copy(idx_hbm.at[pl.ds(base0 + i * _DEG_C, _DEG_C)], idx_v)
        pltpu.sync_copy(ones_v, hist_sp.at[idx_v], add=True)
        return 0

    lax.fori_loop(0, _DEG_ITERS, body, 0)
    plsc.subcore_barrier()
    pltpu.sync_copy(hist_sp.at[pl.ds(s * _HIST_PER_SUB, _HIST_PER_SUB)],
                    out_hbm.at[c, pl.ds(s * _HIST_PER_SUB, _HIST_PER_SUB)])


def _sc_degrees(idx_flat):
    mesh = plsc.VectorSubcoreMesh(core_axis_name="c", subcore_axis_name="s",
                                  num_cores=NC, num_subcores=NS)
    return pl.kernel(
        _sc_degree_body,
        out_type=jax.ShapeDtypeStruct((NC, HIST), jnp.float32),
        mesh=mesh,
        scratch_types=[
            pltpu.VMEM_SHARED((HIST,), jnp.float32),
            pltpu.VMEM((_DEG_C,), jnp.int32),
            pltpu.VMEM((_DEG_C,), jnp.float32),
            pltpu.VMEM((_HIST_PER_SUB,), jnp.float32),
        ],
    )(idx_flat)


_AGG_PER_SUB = E // NS
_AGG_C = 80
_AGG_ITERS = _AGG_PER_SUB // _AGG_C
N_PAD = 10240
_ROWS_PER_SUB = N_PAD // NS
_ZBLK = 32


def _sc_agg_body(table_hbm, src_hbm, dst_hbm, out_hbm,
                 acc_sp, sidx_v, didx_v, rows_v, zero_v, gsem):
    c = lax.axis_index("c")
    s = lax.axis_index("s")
    zeros16 = jnp.zeros((L,), jnp.float32)
    for r in range(_ZBLK):
        for k in range(H // L):
            zero_v[r, pl.ds(k * L, L)] = zeros16

    row0 = s * _ROWS_PER_SUB

    def zbody(i, _):
        pltpu.sync_copy(zero_v, acc_sp.at[pl.ds(row0 + i * _ZBLK, _ZBLK)])
        return 0

    lax.fori_loop(0, _ROWS_PER_SUB // _ZBLK, zbody, 0)
    plsc.subcore_barrier()

    base0 = c * E + s * _AGG_PER_SUB

    def body(i, _):
        b = base0 + i * _AGG_C
        pltpu.sync_copy(src_hbm.at[pl.ds(b, _AGG_C)], sidx_v)
        pltpu.sync_copy(dst_hbm.at[pl.ds(b, _AGG_C)], didx_v)
        pltpu.async_copy(table_hbm.at[sidx_v], rows_v, gsem).wait()
        pltpu.sync_copy(rows_v, acc_sp.at[didx_v], add=True)
        return 0

    lax.fori_loop(0, _AGG_ITERS, body, 0)
    plsc.subcore_barrier()
    pltpu.sync_copy(acc_sp.at[pl.ds(row0, _ROWS_PER_SUB)],
                    out_hbm.at[c, pl.ds(row0, _ROWS_PER_SUB)])


def _sc_aggregate(table, src_all, dst_all):
    mesh = plsc.VectorSubcoreMesh(core_axis_name="c", subcore_axis_name="s",
                                  num_cores=NC, num_subcores=NS)
    return pl.kernel(
        _sc_agg_body,
        out_type=jax.ShapeDtypeStruct((NC, N_PAD, H), jnp.float32),
        mesh=mesh,
        scratch_types=[
            pltpu.VMEM_SHARED((N_PAD, H), jnp.float32),
            pltpu.VMEM((_AGG_C,), jnp.int32),
            pltpu.VMEM((_AGG_C,), jnp.int32),
            pltpu.VMEM((_AGG_C, H), jnp.float32),
            pltpu.VMEM((_ZBLK, H), jnp.float32),
            pltpu.SemaphoreType.DMA,
        ],
    )(table, src_all, dst_all)


def _decoder_matmul_kernel(a_ref, b_ref, o_ref):
    o_ref[...] = lax.dot_general(
        a_ref[...], b_ref[...], (((1,), (1,)), ((), ())),
        preferred_element_type=jnp.float32)


def _decoder_matmul(a, b, bm=512, bn=512):
    m, k = a.shape
    n = b.shape[0]
    grid = (pl.cdiv(m, bm), pl.cdiv(n, bn))
    return pl.pallas_call(
        _decoder_matmul_kernel,
        grid=grid,
        in_specs=[
            pl.BlockSpec((bm, k), lambda i, j: (i, 0)),
            pl.BlockSpec((bn, k), lambda i, j: (j, 0)),
        ],
        out_specs=pl.BlockSpec((bm, bn), lambda i, j: (i, j)),
        out_shape=jax.ShapeDtypeStruct((m, n), jnp.float32),
    )(a, b)


def _bn_prelu(v, gamma, beta, a):
    v = gamma * v + beta
    return jnp.where(v >= 0, v, a * v)


def _sem_att(z, W1, b1, w2):
    w = jnp.tanh(z @ W1.T + b1) @ w2
    beta = jax.nn.softmax(w.mean(0))
    return (beta[None, :, None] * z).sum(1)


def kernel(x_drug, x_disease, edge_dr2di, edge_di2dr,
           W_drug_lin, b_drug_lin, W_dis_lin, b_dis_lin,
           e1_W_dr2di, e1_b_dr2di, e1_W_di2dr, e1_b_di2dr, e1_gamma, e1_beta, e1_prelu,
           e2_W_dr2di, e2_b_dr2di, e2_W_di2dr, e2_b_di2dr, e2_gamma, e2_beta, e2_prelu,
           att_dr_W1, att_dr_b1, att_dr_w2, att_di_W1, att_di_b1, att_di_w2,
           W_R, W_D):
    src_all = jnp.concatenate([edge_dr2di[0], edge_di2dr[0] + N_DRUG])
    dst_all = jnp.concatenate([edge_dr2di[1], edge_di2dr[1]])
    dst_off = jnp.concatenate([edge_dr2di[1], edge_di2dr[1] + N_DIS])
    deg_idx = jnp.concatenate([src_all, dst_off])

    hists = _sc_degrees(deg_idx)
    inv_src = lax.rsqrt(jnp.maximum(hists[0, :N_ALL], 1.0))[:, None]
    inv_dst = lax.rsqrt(jnp.maximum(hists[1, :N_ALL], 1.0))[:, None]
    inv_dst_di = inv_dst[:N_DIS]
    inv_dst_dr = inv_dst[N_DIS:]

    h_dr = x_drug @ W_drug_lin.T + b_drug_lin
    h_di = x_disease @ W_dis_lin.T + b_dis_lin

    table1 = jnp.concatenate([h_dr, h_di]) * inv_src
    agg1 = _sc_aggregate(table1, src_all, dst_all)
    new_di = (agg1[0, :N_DIS] * inv_dst_di) @ e1_W_dr2di.T + e1_b_dr2di
    new_dr = (agg1[1, :N_DRUG] * inv_dst_dr) @ e1_W_di2dr.T + e1_b_di2dr
    h_dr1 = _bn_prelu(new_dr, e1_gamma, e1_beta, e1_prelu)
    h_di1 = _bn_prelu(new_di, e1_gamma, e1_beta, e1_prelu)

    table2 = jnp.concatenate([h_dr1, h_di1]) * inv_src
    agg2 = _sc_aggregate(table2, src_all, dst_all)
    new_di = (agg2[0, :N_DIS] * inv_dst_di) @ e2_W_dr2di.T + e2_b_dr2di
    new_dr = (agg2[1, :N_DRUG] * inv_dst_dr) @ e2_W_di2dr.T + e2_b_di2dr
    h_dr2 = _bn_prelu(new_dr, e2_gamma, e2_beta, e2_prelu)
    h_di2 = _bn_prelu(new_di, e2_gamma, e2_beta, e2_prelu)

    z_dr = jnp.stack([h_dr, h_dr1, h_dr2], axis=1)
    z_di = jnp.stack([h_di, h_di1, h_di2], axis=1)
    drug_f = _sem_att(z_dr, att_dr_W1, att_dr_b1, att_dr_w2)
    dis_f = _sem_att(z_di, att_di_W1, att_di_b1, att_di_w2)

    a = drug_f @ (W_R.T @ W_D)
    return _decoder_matmul(a, dis_f)


# speedup vs baseline: 4.2369x; 1.4718x over previous
import functools

import jax
import jax.numpy as jnp
from jax import lax
from jax.experimental import pallas as pl
from jax.experimental.pallas import tpu as pltpu
from jax.experimental.pallas import tpu_sc as plsc

N_DRUG = 10000
N_DIS = 10000
N_ALL = N_DRUG + N_DIS
E = 320000
H = 128

NC = 2
NS = 16
L = 16

HIST = 20480
_DEG_PER_CORE = 2 * E
_DEG_PER_SUB = _DEG_PER_CORE // NS
_DEG_C = 80
_DEG_ITERS = _DEG_PER_SUB // _DEG_C
_HIST_PER_SUB = HIST // NS


def _sc_degree_body(idx_hbm, out_hbm, hist_sp, idx_v, ones_v, zero_v):
    c = lax.axis_index("c")
    s = lax.axis_index("s")
    zeros16 = jnp.zeros((L,), jnp.float32)
    ones16 = jnp.ones((L,), jnp.float32)
    def fill_zero(i, _):
        zero_v[pl.ds(i * L, L)] = zeros16
        return 0
    lax.fori_loop(0, _HIST_PER_SUB // L, fill_zero, 0)
    for k in range(_DEG_C // L):
        ones_v[pl.ds(k * L, L)] = ones16
    pltpu.sync_copy(zero_v, hist_sp.at[pl.ds(s * _HIST_PER_SUB, _HIST_PER_SUB)])
    plsc.subcore_barrier()

    base0 = c * _DEG_PER_CORE + s * _DEG_PER_SUB

    def body(i, _):
        pltpu.sync_copy(idx_hbm.at[pl.ds(base0 + i * _DEG_C, _DEG_C)], idx_v)
        pltpu.sync_copy(ones_v, hist_sp.at[idx_v], add=True)
        return 0

    lax.fori_loop(0, _DEG_ITERS, body, 0)
    plsc.subcore_barrier()
    pltpu.sync_copy(hist_sp.at[pl.ds(s * _HIST_PER_SUB, _HIST_PER_SUB)],
                    out_hbm.at[c, pl.ds(s * _HIST_PER_SUB, _HIST_PER_SUB)])


def _sc_degrees(idx_flat):
    mesh = plsc.VectorSubcoreMesh(core_axis_name="c", subcore_axis_name="s",
                                  num_cores=NC, num_subcores=NS)
    return pl.kernel(
        _sc_degree_body,
        out_type=jax.ShapeDtypeStruct((NC, HIST), jnp.float32),
        mesh=mesh,
        scratch_types=[
            pltpu.VMEM_SHARED((HIST,), jnp.float32),
            pltpu.VMEM((_DEG_C,), jnp.int32),
            pltpu.VMEM((_DEG_C,), jnp.float32),
            pltpu.VMEM((_HIST_PER_SUB,), jnp.float32),
        ],
    )(idx_flat)


_AGG_PER_SUB = E // NS
_AGG_C = 128
_AGG_ITERS = _AGG_PER_SUB // _AGG_C
_AGG_TAIL = _AGG_PER_SUB - _AGG_ITERS * _AGG_C
N_PAD = 10240
_ROWS_PER_SUB = N_PAD // NS
_ZBLK = 32


def _sc_agg_body(table_hbm, src_hbm, dst_hbm, out_hbm,
                 acc_sp, sidx, didx, rows, tidx_s, tidx_d, trows,
                 zero_v, gsem0, gsem1, tsem):
    c = lax.axis_index("c")
    s = lax.axis_index("s")
    zeros16 = jnp.zeros((L,), jnp.float32)
    for r in range(_ZBLK):
        for k in range(H // L):
            zero_v[r, pl.ds(k * L, L)] = zeros16

    row0 = s * _ROWS_PER_SUB

    def zbody(i, _):
        pltpu.sync_copy(zero_v, acc_sp.at[pl.ds(row0 + i * _ZBLK, _ZBLK)])
        return 0

    lax.fori_loop(0, _ROWS_PER_SUB // _ZBLK, zbody, 0)
    plsc.subcore_barrier()

    base0 = c * E + s * _AGG_PER_SUB
    sidx0, sidx1 = sidx.at[0], sidx.at[1]
    didx0, didx1 = didx.at[0], didx.at[1]
    rows0, rows1 = rows.at[0], rows.at[1]
    g0, g1 = gsem0, gsem1

    pltpu.sync_copy(src_hbm.at[pl.ds(base0, _AGG_C)], sidx0)
    pltpu.sync_copy(dst_hbm.at[pl.ds(base0, _AGG_C)], didx0)
    pltpu.async_copy(table_hbm.at[sidx0], rows0, g0)

    def body2(g, _):
        for (i_off, (sa, da, ra, ga), (sb, db, rb, gb)) in (
                (0, (sidx0, didx0, rows0, g0), (sidx1, didx1, rows1, g1)),
                (1, (sidx1, didx1, rows1, g1), (sidx0, didx0, rows0, g0))):
            i = 2 * g + i_off
            nxt = i + 1

            @pl.when(nxt < _AGG_ITERS)
            def _():
                b = base0 + nxt * _AGG_C
                pltpu.sync_copy(src_hbm.at[pl.ds(b, _AGG_C)], sb)
                pltpu.sync_copy(dst_hbm.at[pl.ds(b, _AGG_C)], db)
                pltpu.async_copy(table_hbm.at[sb], rb, gb)

            pltpu.make_async_copy(table_hbm.at[sa], ra, ga).wait()
            pltpu.sync_copy(ra, acc_sp.at[da], add=True)
        return 0

    lax.fori_loop(0, (_AGG_ITERS + 1) // 2, body2, 0)
    bt = base0 + _AGG_ITERS * _AGG_C
    pltpu.sync_copy(src_hbm.at[pl.ds(bt, _AGG_TAIL)], tidx_s)
    pltpu.sync_copy(dst_hbm.at[pl.ds(bt, _AGG_TAIL)], tidx_d)
    pltpu.async_copy(table_hbm.at[tidx_s], trows, tsem).wait()
    pltpu.sync_copy(trows, acc_sp.at[tidx_d], add=True)

    plsc.subcore_barrier()
    pltpu.sync_copy(acc_sp.at[pl.ds(row0, _ROWS_PER_SUB)],
                    out_hbm.at[c, pl.ds(row0, _ROWS_PER_SUB)])


def _sc_aggregate(table, src_all, dst_all):
    mesh = plsc.VectorSubcoreMesh(core_axis_name="c", subcore_axis_name="s",
                                  num_cores=NC, num_subcores=NS)
    return pl.kernel(
        _sc_agg_body,
        out_type=jax.ShapeDtypeStruct((NC, N_PAD, H), jnp.float32),
        mesh=mesh,
        scratch_types=[
            pltpu.VMEM_SHARED((N_PAD, H), jnp.float32),
            pltpu.VMEM((2, _AGG_C), jnp.int32),
            pltpu.VMEM((2, _AGG_C), jnp.int32),
            pltpu.VMEM((2, _AGG_C, H), jnp.float32),
            pltpu.VMEM((_AGG_TAIL,), jnp.int32),
            pltpu.VMEM((_AGG_TAIL,), jnp.int32),
            pltpu.VMEM((_AGG_TAIL, H), jnp.float32),
            pltpu.VMEM((_ZBLK, H), jnp.float32),
            pltpu.SemaphoreType.DMA,
            pltpu.SemaphoreType.DMA,
            pltpu.SemaphoreType.DMA,
        ],
    )(table, src_all, dst_all)


def _decoder_matmul_kernel(a_ref, b_ref, o_ref):
    o_ref[...] = lax.dot_general(
        a_ref[...], b_ref[...], (((1,), (1,)), ((), ())),
        preferred_element_type=jnp.float32)


def _decoder_matmul(a, b, bm=512, bn=512):
    m, k = a.shape
    n = b.shape[0]
    grid = (pl.cdiv(m, bm), pl.cdiv(n, bn))
    return pl.pallas_call(
        _decoder_matmul_kernel,
        grid=grid,
        in_specs=[
            pl.BlockSpec((bm, k), lambda i, j: (i, 0)),
            pl.BlockSpec((bn, k), lambda i, j: (j, 0)),
        ],
        out_specs=pl.BlockSpec((bm, bn), lambda i, j: (i, j)),
        out_shape=jax.ShapeDtypeStruct((m, n), jnp.float32),
    )(a, b)


def _bn_prelu(v, gamma, beta, a):
    v = gamma * v + beta
    return jnp.where(v >= 0, v, a * v)


def _sem_att(z, W1, b1, w2):
    w = jnp.tanh(z @ W1.T + b1) @ w2
    beta = jax.nn.softmax(w.mean(0))
    return (beta[None, :, None] * z).sum(1)


def kernel(x_drug, x_disease, edge_dr2di, edge_di2dr,
           W_drug_lin, b_drug_lin, W_dis_lin, b_dis_lin,
           e1_W_dr2di, e1_b_dr2di, e1_W_di2dr, e1_b_di2dr, e1_gamma, e1_beta, e1_prelu,
           e2_W_dr2di, e2_b_dr2di, e2_W_di2dr, e2_b_di2dr, e2_gamma, e2_beta, e2_prelu,
           att_dr_W1, att_dr_b1, att_dr_w2, att_di_W1, att_di_b1, att_di_w2,
           W_R, W_D):
    src_all = jnp.concatenate([edge_dr2di[0], edge_di2dr[0] + N_DRUG])
    dst_all = jnp.concatenate([edge_dr2di[1], edge_di2dr[1]])
    dst_off = jnp.concatenate([edge_dr2di[1], edge_di2dr[1] + N_DIS])
    deg_idx = jnp.concatenate([src_all, dst_off])

    hists = _sc_degrees(deg_idx)
    inv_src = lax.rsqrt(jnp.maximum(hists[0, :N_ALL], 1.0))[:, None]
    inv_dst = lax.rsqrt(jnp.maximum(hists[1, :N_ALL], 1.0))[:, None]
    inv_dst_di = inv_dst[:N_DIS]
    inv_dst_dr = inv_dst[N_DIS:]

    h_dr = x_drug @ W_drug_lin.T + b_drug_lin
    h_di = x_disease @ W_dis_lin.T + b_dis_lin

    table1 = jnp.concatenate([h_dr, h_di]) * inv_src
    agg1 = _sc_aggregate(table1, src_all, dst_all)
    new_di = (agg1[0, :N_DIS] * inv_dst_di) @ e1_W_dr2di.T + e1_b_dr2di
    new_dr = (agg1[1, :N_DRUG] * inv_dst_dr) @ e1_W_di2dr.T + e1_b_di2dr
    h_dr1 = _bn_prelu(new_dr, e1_gamma, e1_beta, e1_prelu)
    h_di1 = _bn_prelu(new_di, e1_gamma, e1_beta, e1_prelu)

    table2 = jnp.concatenate([h_dr1, h_di1]) * inv_src
    agg2 = _sc_aggregate(table2, src_all, dst_all)
    new_di = (agg2[0, :N_DIS] * inv_dst_di) @ e2_W_dr2di.T + e2_b_dr2di
    new_dr = (agg2[1, :N_DRUG] * inv_dst_dr) @ e2_W_di2dr.T + e2_b_di2dr
    h_dr2 = _bn_prelu(new_dr, e2_gamma, e2_beta, e2_prelu)
    h_di2 = _bn_prelu(new_di, e2_gamma, e2_beta, e2_prelu)

    z_dr = jnp.stack([h_dr, h_dr1, h_dr2], axis=1)
    z_di = jnp.stack([h_di, h_di1, h_di2], axis=1)
    drug_f = _sem_att(z_dr, att_dr_W1, att_dr_b1, att_dr_w2)
    dis_f = _sem_att(z_di, att_di_W1, att_di_b1, att_di_w2)

    a = drug_f @ (W_R.T @ W_D)
    return _decoder_matmul(a, dis_f)


# speedup vs baseline: 4.5276x; 1.0686x over previous
import functools

import jax
import jax.numpy as jnp
from jax import lax
from jax.experimental import pallas as pl
from jax.experimental.pallas import tpu as pltpu
from jax.experimental.pallas import tpu_sc as plsc

N_DRUG = 10000
N_DIS = 10000
N_ALL = N_DRUG + N_DIS
E = 320000
H = 128

NC = 2
NS = 16
L = 16

HIST = 20480
_DEG_C = 80
_DEG_ROWS = 16384
_DEG_ROWS_SUB = _DEG_ROWS // (NC * NS)
_DEG_FIRE = 8
_HIST_PER_SUB = HIST // NS


def _sc_degree_body(idx_hbm, out_hbm, hist_sp, idx_blk, ones_v, zero_v, sem):
    c = lax.axis_index("c")
    s = lax.axis_index("s")
    zeros16 = jnp.zeros((L,), jnp.float32)
    ones16 = jnp.ones((L,), jnp.float32)
    def fill_zero(i, _):
        zero_v[pl.ds(i * L, L)] = zeros16
        return 0
    lax.fori_loop(0, _HIST_PER_SUB // L, fill_zero, 0)
    for k in range(_DEG_C // L):
        ones_v[pl.ds(k * L, L)] = ones16
    pltpu.sync_copy(zero_v, hist_sp.at[pl.ds(s * _HIST_PER_SUB, _HIST_PER_SUB)])

    row0 = c * (_DEG_ROWS // 2) + s * _DEG_ROWS_SUB
    pltpu.sync_copy(idx_hbm.at[pl.ds(row0, _DEG_ROWS_SUB)], idx_blk)
    plsc.subcore_barrier()

    def body(g, _):
        for k in range(_DEG_FIRE):
            pltpu.async_copy(ones_v, hist_sp.at[idx_blk.at[g * _DEG_FIRE + k]],
                             sem, add=True)
        for k in range(_DEG_FIRE):
            pltpu.make_async_copy(ones_v, hist_sp.at[idx_blk.at[0]], sem).wait()
        return 0

    lax.fori_loop(0, _DEG_ROWS_SUB // _DEG_FIRE, body, 0)
    plsc.subcore_barrier()
    pltpu.sync_copy(hist_sp.at[pl.ds(s * _HIST_PER_SUB, _HIST_PER_SUB)],
                    out_hbm.at[c, pl.ds(s * _HIST_PER_SUB, _HIST_PER_SUB)])


def _sc_degrees(idx_blocks):
    mesh = plsc.VectorSubcoreMesh(core_axis_name="c", subcore_axis_name="s",
                                  num_cores=NC, num_subcores=NS)
    return pl.kernel(
        _sc_degree_body,
        out_type=jax.ShapeDtypeStruct((NC, HIST), jnp.float32),
        mesh=mesh,
        scratch_types=[
            pltpu.VMEM_SHARED((HIST,), jnp.float32),
            pltpu.VMEM((_DEG_ROWS_SUB, _DEG_C), jnp.int32),
            pltpu.VMEM((_DEG_C,), jnp.float32),
            pltpu.VMEM((_HIST_PER_SUB,), jnp.float32),
            pltpu.SemaphoreType.DMA,
        ],
    )(idx_blocks)


_AGG_C = 128
_AGG_CHUNKS = 157
N_PAD = 10240
_ROWS_PER_SUB = N_PAD // NS
_ZBLK = 32


def _sc_agg_body(table_hbm, src_hbm, dst_hbm, out_hbm,
                 acc_sp, sidx, didx, rows, zero_v,
                 g0, g1, s0, s1, i0, i1):
    c = lax.axis_index("c")
    s = lax.axis_index("s")
    zeros16 = jnp.zeros((L,), jnp.float32)
    for r in range(_ZBLK):
        for k in range(H // L):
            zero_v[r, pl.ds(k * L, L)] = zeros16

    row0 = s * _ROWS_PER_SUB

    def zbody(i, _):
        pltpu.sync_copy(zero_v, acc_sp.at[pl.ds(row0 + i * _ZBLK, _ZBLK)])
        return 0

    lax.fori_loop(0, _ROWS_PER_SUB // _ZBLK, zbody, 0)
    plsc.subcore_barrier()

    gsems = (g0, g1)
    ssems = (s0, s1)
    isems = (i0, i1)

    def idx_start(j, b):
        pltpu.async_copy(src_hbm.at[c, s, j], sidx.at[b], isems[b])
        pltpu.async_copy(dst_hbm.at[c, s, j], didx.at[b], isems[b])

    def idx_wait(j, b):
        pltpu.make_async_copy(src_hbm.at[c, s, j], sidx.at[b], isems[b]).wait()
        pltpu.make_async_copy(dst_hbm.at[c, s, j], didx.at[b], isems[b]).wait()

    idx_start(0, 0)
    idx_start(1, 1)
    idx_wait(0, 0)
    pltpu.async_copy(table_hbm.at[sidx.at[0]], rows.at[0], gsems[0])

    def group(g, _):
        for b in range(2):
            bb = 1 - b
            j = 2 * g + b
            nxt = j + 1

            @pl.when(nxt < _AGG_CHUNKS)
            def _():
                idx_wait(nxt, bb)
            pltpu.make_async_copy(table_hbm.at[sidx.at[b]], rows.at[b],
                                  gsems[b]).wait()
            pltpu.async_copy(rows.at[b], acc_sp.at[didx.at[b]], ssems[b],
                             add=True)

            @pl.when(nxt < _AGG_CHUNKS)
            def _():
                pltpu.async_copy(table_hbm.at[sidx.at[bb]], rows.at[bb],
                                 gsems[bb])

            @pl.when(nxt + 1 < _AGG_CHUNKS)
            def _():
                pltpu.make_async_copy(rows.at[b], acc_sp.at[didx.at[b]],
                                      ssems[b]).wait()
                idx_start(nxt + 1, b)
        return 0

    lax.fori_loop(0, _AGG_CHUNKS // 2, group, 0)
    pltpu.make_async_copy(table_hbm.at[sidx.at[0]], rows.at[0],
                          gsems[0]).wait()
    pltpu.async_copy(rows.at[0], acc_sp.at[didx.at[0]], ssems[0], add=True)
    pltpu.make_async_copy(rows.at[0], acc_sp.at[didx.at[0]], ssems[0]).wait()
    pltpu.make_async_copy(rows.at[1], acc_sp.at[didx.at[1]], ssems[1]).wait()

    plsc.subcore_barrier()
    pltpu.sync_copy(acc_sp.at[pl.ds(row0, _ROWS_PER_SUB)],
                    out_hbm.at[c, pl.ds(row0, _ROWS_PER_SUB)])


def _sc_aggregate(table, src_blocks, dst_blocks):
    mesh = plsc.VectorSubcoreMesh(core_axis_name="c", subcore_axis_name="s",
                                  num_cores=NC, num_subcores=NS)
    return pl.kernel(
        _sc_agg_body,
        out_type=jax.ShapeDtypeStruct((NC, N_PAD, H), jnp.float32),
        mesh=mesh,
        scratch_types=[
            pltpu.VMEM_SHARED((N_PAD, H), jnp.float32),
            pltpu.VMEM((2, _AGG_C), jnp.int32),
            pltpu.VMEM((2, _AGG_C), jnp.int32),
            pltpu.VMEM((2, _AGG_C, H), jnp.float32),
            pltpu.VMEM((_ZBLK, H), jnp.float32),
        ] + [pltpu.SemaphoreType.DMA] * 6,
    )(table, src_blocks, dst_blocks)


def _decoder_matmul_kernel(a_ref, b_ref, o_ref):
    o_ref[...] = lax.dot_general(
        a_ref[...], b_ref[...], (((1,), (1,)), ((), ())),
        preferred_element_type=jnp.float32)


def _decoder_matmul(a, b, bm=512, bn=512):
    m, k = a.shape
    n = b.shape[0]
    grid = (pl.cdiv(m, bm), pl.cdiv(n, bn))
    return pl.pallas_call(
        _decoder_matmul_kernel,
        grid=grid,
        in_specs=[
            pl.BlockSpec((bm, k), lambda i, j: (i, 0)),
            pl.BlockSpec((bn, k), lambda i, j: (j, 0)),
        ],
        out_specs=pl.BlockSpec((bm, bn), lambda i, j: (i, j)),
        out_shape=jax.ShapeDtypeStruct((m, n), jnp.float32),
    )(a, b)


def _bn_prelu(v, gamma, beta, a):
    v = gamma * v + beta
    return jnp.where(v >= 0, v, a * v)


def _sem_att(z, W1, b1, w2):
    w = jnp.tanh(z @ W1.T + b1) @ w2
    beta = jax.nn.softmax(w.mean(0))
    return (beta[None, :, None] * z).sum(1)


def _edge_blocks(idx, offset, pad_value):
    per_sub = E // NS
    pad = _AGG_CHUNKS * _AGG_C - per_sub
    blk = idx.reshape(NS, per_sub) + offset
    blk = jnp.pad(blk, ((0, 0), (0, pad)), constant_values=pad_value)
    return blk.reshape(NS, _AGG_CHUNKS, _AGG_C)


def kernel(x_drug, x_disease, edge_dr2di, edge_di2dr,
           W_drug_lin, b_drug_lin, W_dis_lin, b_dis_lin,
           e1_W_dr2di, e1_b_dr2di, e1_W_di2dr, e1_b_di2dr, e1_gamma, e1_beta, e1_prelu,
           e2_W_dr2di, e2_b_dr2di, e2_W_di2dr, e2_b_di2dr, e2_gamma, e2_beta, e2_prelu,
           att_dr_W1, att_dr_b1, att_dr_w2, att_di_W1, att_di_b1, att_di_w2,
           W_R, W_D):
    src_blocks = jnp.stack([
        _edge_blocks(edge_dr2di[0], 0, 0),
        _edge_blocks(edge_di2dr[0], N_DRUG, 0),
    ])
    dst_blocks = jnp.stack([
        _edge_blocks(edge_dr2di[1], 0, N_PAD - 8),
        _edge_blocks(edge_di2dr[1], 0, N_PAD - 8),
    ])

    src_all = jnp.concatenate([edge_dr2di[0], edge_di2dr[0] + N_DRUG])
    dst_off = jnp.concatenate([edge_dr2di[1], edge_di2dr[1] + N_DIS])
    deg_idx = jnp.concatenate([src_all, dst_off])
    deg_pad = _DEG_ROWS * _DEG_C - deg_idx.shape[0]
    deg_idx = jnp.pad(deg_idx, (0, deg_pad), constant_values=N_ALL)
    deg_blocks = deg_idx.reshape(_DEG_ROWS, _DEG_C)

    hists = _sc_degrees(deg_blocks)
    inv_src = lax.rsqrt(jnp.maximum(hists[0, :N_ALL], 1.0))[:, None]
    inv_dst = lax.rsqrt(jnp.maximum(hists[1, :N_ALL], 1.0))[:, None]
    inv_dst_di = inv_dst[:N_DIS]
    inv_dst_dr = inv_dst[N_DIS:]

    h_dr = x_drug @ W_drug_lin.T + b_drug_lin
    h_di = x_disease @ W_dis_lin.T + b_dis_lin

    table1 = jnp.concatenate([h_dr, h_di]) * inv_src
    agg1 = _sc_aggregate(table1, src_blocks, dst_blocks)
    new_di = (agg1[0, :N_DIS] * inv_dst_di) @ e1_W_dr2di.T + e1_b_dr2di
    new_dr = (agg1[1, :N_DRUG] * inv_dst_dr) @ e1_W_di2dr.T + e1_b_di2dr
    h_dr1 = _bn_prelu(new_dr, e1_gamma, e1_beta, e1_prelu)
    h_di1 = _bn_prelu(new_di, e1_gamma, e1_beta, e1_prelu)

    table2 = jnp.concatenate([h_dr1, h_di1]) * inv_src
    agg2 = _sc_aggregate(table2, src_blocks, dst_blocks)
    new_di = (agg2[0, :N_DIS] * inv_dst_di) @ e2_W_dr2di.T + e2_b_dr2di
    new_dr = (agg2[1, :N_DRUG] * inv_dst_dr) @ e2_W_di2dr.T + e2_b_di2dr
    h_dr2 = _bn_prelu(new_dr, e2_gamma, e2_beta, e2_prelu)
    h_di2 = _bn_prelu(new_di, e2_gamma, e2_beta, e2_prelu)

    z_dr = jnp.stack([h_dr, h_dr1, h_dr2], axis=1)
    z_di = jnp.stack([h_di, h_di1, h_di2], axis=1)
    drug_f = _sem_att(z_dr, att_dr_W1, att_dr_b1, att_dr_w2)
    dis_f = _sem_att(z_di, att_di_W1, att_di_b1, att_di_w2)

    a = drug_f @ (W_R.T @ W_D)
    return _decoder_matmul(a.astype(jnp.bfloat16), dis_f.astype(jnp.bfloat16))


# speedup vs baseline: 4.9681x; 1.0973x over previous
import functools

import jax
import jax.numpy as jnp
from jax import lax
from jax.experimental import pallas as pl
from jax.experimental.pallas import tpu as pltpu
from jax.experimental.pallas import tpu_sc as plsc

N_DRUG = 10000
N_DIS = 10000
N_ALL = N_DRUG + N_DIS
E = 320000
H = 128

NC = 2
NS = 16
L = 16

HIST = 20480
_DEG_C = 80
_DEG_ROWS = 16384
_DEG_ROWS_SUB = _DEG_ROWS // (NC * NS)
_DEG_FIRE = 8
_HIST_PER_SUB = HIST // NS


def _sc_degree_body(idx_hbm, out_hbm, hist_sp, idx_blk, ones_v, zero_v, sem):
    c = lax.axis_index("c")
    s = lax.axis_index("s")
    zeros16 = jnp.zeros((L,), jnp.float32)
    ones16 = jnp.ones((L,), jnp.float32)
    def fill_zero(i, _):
        zero_v[pl.ds(i * L, L)] = zeros16
        return 0
    lax.fori_loop(0, _HIST_PER_SUB // L, fill_zero, 0)
    for k in range(_DEG_C // L):
        ones_v[pl.ds(k * L, L)] = ones16
    pltpu.sync_copy(zero_v, hist_sp.at[pl.ds(s * _HIST_PER_SUB, _HIST_PER_SUB)])

    row0 = c * (_DEG_ROWS // 2) + s * _DEG_ROWS_SUB
    pltpu.sync_copy(idx_hbm.at[pl.ds(row0, _DEG_ROWS_SUB)], idx_blk)
    plsc.subcore_barrier()

    def body(g, _):
        for k in range(_DEG_FIRE):
            pltpu.async_copy(ones_v, hist_sp.at[idx_blk.at[g * _DEG_FIRE + k]],
                             sem, add=True)
        for k in range(_DEG_FIRE):
            pltpu.make_async_copy(ones_v, hist_sp.at[idx_blk.at[0]], sem).wait()
        return 0

    lax.fori_loop(0, _DEG_ROWS_SUB // _DEG_FIRE, body, 0)
    plsc.subcore_barrier()
    pltpu.sync_copy(hist_sp.at[pl.ds(s * _HIST_PER_SUB, _HIST_PER_SUB)],
                    out_hbm.at[c, pl.ds(s * _HIST_PER_SUB, _HIST_PER_SUB)])


def _sc_degrees(idx_blocks):
    mesh = plsc.VectorSubcoreMesh(core_axis_name="c", subcore_axis_name="s",
                                  num_cores=NC, num_subcores=NS)
    return pl.kernel(
        _sc_degree_body,
        out_type=jax.ShapeDtypeStruct((NC, HIST), jnp.float32),
        mesh=mesh,
        scratch_types=[
            pltpu.VMEM_SHARED((HIST,), jnp.float32),
            pltpu.VMEM((_DEG_ROWS_SUB, _DEG_C), jnp.int32),
            pltpu.VMEM((_DEG_C,), jnp.float32),
            pltpu.VMEM((_HIST_PER_SUB,), jnp.float32),
            pltpu.SemaphoreType.DMA,
        ],
    )(idx_blocks)


_AGG_C = 104
_AGG_CHUNKS = 193
N_PAD = 10112
_ROWS_PER_SUB = N_PAD // NS
_ZBLK = 8


def _sc_agg_body(table_hbm, src_hbm, dst_hbm, out_hbm,
                 acc_sp, sidx, didx, rows,
                 g0, g1, g2, s0, s1, s2, i0, i1, i2):
    c = lax.axis_index("c")
    s = lax.axis_index("s")
    zeros16 = jnp.zeros((L,), jnp.float32)
    for r in range(_ZBLK):
        for k in range(H // L):
            rows[0, r, pl.ds(k * L, L)] = zeros16

    row0 = s * _ROWS_PER_SUB
    zsrc = rows.at[0].at[pl.ds(0, _ZBLK)]

    def zbody(i, _):
        pltpu.sync_copy(zsrc, acc_sp.at[pl.ds(row0 + i * _ZBLK, _ZBLK)])
        return 0

    lax.fori_loop(0, _ROWS_PER_SUB // _ZBLK, zbody, 0)
    plsc.subcore_barrier()

    gsems = (g0, g1, g2)
    ssems = (s0, s1, s2)
    isems = (i0, i1, i2)

    def idx_start(j, b):
        pltpu.async_copy(src_hbm.at[c, s, j], sidx.at[b], isems[b])
        pltpu.async_copy(dst_hbm.at[c, s, j], didx.at[b], isems[b])

    def idx_wait(j, b):
        pltpu.make_async_copy(src_hbm.at[c, s, j], sidx.at[b], isems[b]).wait()
        pltpu.make_async_copy(dst_hbm.at[c, s, j], didx.at[b], isems[b]).wait()

    idx_start(0, 0)
    idx_start(1, 1)
    idx_wait(0, 0)
    pltpu.async_copy(table_hbm.at[sidx.at[0]], rows.at[0], gsems[0])

    def group(g, _):
        for b in range(3):
            bn = (b + 1) % 3
            bp = (b - 1) % 3
            j = 3 * g + b
            nxt = j + 1

            @pl.when(nxt < _AGG_CHUNKS)
            def _():
                idx_wait(nxt, bn)
                pltpu.async_copy(table_hbm.at[sidx.at[bn]], rows.at[bn],
                                 gsems[bn])
            pltpu.make_async_copy(table_hbm.at[sidx.at[b]], rows.at[b],
                                  gsems[b]).wait()
            pltpu.async_copy(rows.at[b], acc_sp.at[didx.at[b]], ssems[b],
                             add=True)

            @pl.when(j >= 1)
            def _():
                pltpu.make_async_copy(rows.at[bp], acc_sp.at[didx.at[bp]],
                                      ssems[bp]).wait()

            @pl.when(nxt + 1 < _AGG_CHUNKS)
            def _():
                idx_start(nxt + 1, bp)
        return 0

    lax.fori_loop(0, _AGG_CHUNKS // 3, group, 0)
    pltpu.make_async_copy(table_hbm.at[sidx.at[0]], rows.at[0],
                          gsems[0]).wait()
    pltpu.async_copy(rows.at[0], acc_sp.at[didx.at[0]], ssems[0], add=True)
    pltpu.make_async_copy(rows.at[2], acc_sp.at[didx.at[2]], ssems[2]).wait()
    pltpu.make_async_copy(rows.at[0], acc_sp.at[didx.at[0]], ssems[0]).wait()

    plsc.subcore_barrier()
    pltpu.sync_copy(acc_sp.at[pl.ds(row0, _ROWS_PER_SUB)],
                    out_hbm.at[c, pl.ds(row0, _ROWS_PER_SUB)])


def _sc_aggregate(table, src_blocks, dst_blocks):
    mesh = plsc.VectorSubcoreMesh(core_axis_name="c", subcore_axis_name="s",
                                  num_cores=NC, num_subcores=NS)
    return pl.kernel(
        _sc_agg_body,
        out_type=jax.ShapeDtypeStruct((NC, N_PAD, H), jnp.float32),
        mesh=mesh,
        scratch_types=[
            pltpu.VMEM_SHARED((N_PAD, H), jnp.float32),
            pltpu.VMEM((3, _AGG_C), jnp.int32),
            pltpu.VMEM((3, _AGG_C), jnp.int32),
            pltpu.VMEM((3, _AGG_C, H), jnp.float32),
        ] + [pltpu.SemaphoreType.DMA] * 9,
    )(table, src_blocks, dst_blocks)


def _decoder_matmul_kernel(a_ref, b_ref, o_ref):
    o_ref[...] = lax.dot_general(
        a_ref[...], b_ref[...], (((1,), (1,)), ((), ())),
        preferred_element_type=jnp.float32)


def _decoder_matmul(a, b, bm=512, bn=512):
    m, k = a.shape
    n = b.shape[0]
    grid = (pl.cdiv(m, bm), pl.cdiv(n, bn))
    return pl.pallas_call(
        _decoder_matmul_kernel,
        grid=grid,
        in_specs=[
            pl.BlockSpec((bm, k), lambda i, j: (i, 0)),
            pl.BlockSpec((bn, k), lambda i, j: (j, 0)),
        ],
        out_specs=pl.BlockSpec((bm, bn), lambda i, j: (i, j)),
        out_shape=jax.ShapeDtypeStruct((m, n), jnp.float32),
    )(a, b)


def _bn_prelu(v, gamma, beta, a):
    v = gamma * v + beta
    return jnp.where(v >= 0, v, a * v)


def _sem_att(z, W1, b1, w2):
    w = jnp.tanh(z @ W1.T + b1) @ w2
    beta = jax.nn.softmax(w.mean(0))
    return (beta[None, :, None] * z).sum(1)


def _edge_blocks(idx, offset, pad_value):
    per_sub = E // NS
    pad = _AGG_CHUNKS * _AGG_C - per_sub
    blk = idx.reshape(NS, per_sub) + offset
    blk = jnp.pad(blk, ((0, 0), (0, pad)), constant_values=pad_value)
    return blk.reshape(NS, _AGG_CHUNKS, _AGG_C)


def kernel(x_drug, x_disease, edge_dr2di, edge_di2dr,
           W_drug_lin, b_drug_lin, W_dis_lin, b_dis_lin,
           e1_W_dr2di, e1_b_dr2di, e1_W_di2dr, e1_b_di2dr, e1_gamma, e1_beta, e1_prelu,
           e2_W_dr2di, e2_b_dr2di, e2_W_di2dr, e2_b_di2dr, e2_gamma, e2_beta, e2_prelu,
           att_dr_W1, att_dr_b1, att_dr_w2, att_di_W1, att_di_b1, att_di_w2,
           W_R, W_D):
    src_blocks = jnp.stack([
        _edge_blocks(edge_dr2di[0], 0, 0),
        _edge_blocks(edge_di2dr[0], N_DRUG, 0),
    ])
    dst_blocks = jnp.stack([
        _edge_blocks(edge_dr2di[1], 0, N_PAD - 8),
        _edge_blocks(edge_di2dr[1], 0, N_PAD - 8),
    ])

    src_all = jnp.concatenate([edge_dr2di[0], edge_di2dr[0] + N_DRUG])
    dst_off = jnp.concatenate([edge_dr2di[1], edge_di2dr[1] + N_DIS])
    deg_idx = jnp.concatenate([src_all, dst_off])
    deg_pad = _DEG_ROWS * _DEG_C - deg_idx.shape[0]
    deg_idx = jnp.pad(deg_idx, (0, deg_pad), constant_values=N_ALL)
    deg_blocks = deg_idx.reshape(_DEG_ROWS, _DEG_C)

    hists = _sc_degrees(deg_blocks)
    inv_src = lax.rsqrt(jnp.maximum(hists[0, :N_ALL], 1.0))[:, None]
    inv_dst = lax.rsqrt(jnp.maximum(hists[1, :N_ALL], 1.0))[:, None]
    inv_dst_di = inv_dst[:N_DIS]
    inv_dst_dr = inv_dst[N_DIS:]

    h_dr = x_drug @ W_drug_lin.T + b_drug_lin
    h_di = x_disease @ W_dis_lin.T + b_dis_lin

    table1 = jnp.concatenate([h_dr, h_di]) * inv_src
    agg1 = _sc_aggregate(table1, src_blocks, dst_blocks)
    new_di = (agg1[0, :N_DIS] * inv_dst_di) @ e1_W_dr2di.T + e1_b_dr2di
    new_dr = (agg1[1, :N_DRUG] * inv_dst_dr) @ e1_W_di2dr.T + e1_b_di2dr
    h_dr1 = _bn_prelu(new_dr, e1_gamma, e1_beta, e1_prelu)
    h_di1 = _bn_prelu(new_di, e1_gamma, e1_beta, e1_prelu)

    table2 = jnp.concatenate([h_dr1, h_di1]) * inv_src
    agg2 = _sc_aggregate(table2, src_blocks, dst_blocks)
    new_di = (agg2[0, :N_DIS] * inv_dst_di) @ e2_W_dr2di.T + e2_b_dr2di
    new_dr = (agg2[1, :N_DRUG] * inv_dst_dr) @ e2_W_di2dr.T + e2_b_di2dr
    h_dr2 = _bn_prelu(new_dr, e2_gamma, e2_beta, e2_prelu)
    h_di2 = _bn_prelu(new_di, e2_gamma, e2_beta, e2_prelu)

    z_dr = jnp.stack([h_dr, h_dr1, h_dr2], axis=1)
    z_di = jnp.stack([h_di, h_di1, h_di2], axis=1)
    drug_f = _sem_att(z_dr, att_dr_W1, att_dr_b1, att_dr_w2)
    dis_f = _sem_att(z_di, att_di_W1, att_di_b1, att_di_w2)

    a = drug_f @ (W_R.T @ W_D)
    return _decoder_matmul(a.astype(jnp.bfloat16), dis_f.astype(jnp.bfloat16))
